# baseline (speedup 1.0000x reference)
import functools

import jax
import jax.numpy as jnp
from jax import lax
from jax.experimental import pallas as pl
from jax.experimental.pallas import tpu as pltpu
from jax.experimental.pallas import tpu_sc as plsc

E = 8
K = 2
D = 1024
FF = 512
T = 2048
N = T * K
M_TILE = 256
NT = N // M_TILE
G = NT + E - 1
TB = 256
NB = T // TB

NC = 2
NS = 16
NW = NC * NS
TPT = T // NW
DW = D // 2


def _pack_pairs(lo_f32, hi_f32):
    lob = lax.bitcast_convert_type(lo_f32, jnp.int32)
    hib = lax.bitcast_convert_type(hi_f32, jnp.int32)
    lo16 = lax.shift_right_logical(lob + 0x8000, 16)
    hi16 = (hib + 0x8000) & jnp.int32(-65536)
    return hi16 | lo16


def _router_body(x_ref, gw_ref, s0_ref, s1_ref, w0_ref, w1_ref, counts_ref,
                 xp_ref):
    x = x_ref[...]
    xp_ref[...] = _pack_pairs(x[:, :DW], x[:, DW:])
    gw = gw_ref[...]
    logits = lax.dot_general(x, gw, (((1,), (1,)), ((), ())),
                             preferred_element_type=jnp.float32)
    m = jnp.max(logits, axis=1, keepdims=True)
    ex = jnp.exp(logits - m)
    p = ex / jnp.sum(ex, axis=1, keepdims=True)
    lane = lax.broadcasted_iota(jnp.int32, p.shape, 1)
    m1 = jnp.max(p, axis=1, keepdims=True)
    a1 = jnp.min(jnp.where(p == m1, lane, E), axis=1, keepdims=True)
    p2 = jnp.where(lane == a1, -jnp.inf, p)
    m2 = jnp.max(p2, axis=1, keepdims=True)
    a2 = jnp.min(jnp.where(p2 == m2, lane, E), axis=1, keepdims=True)
    denom = m1 + m2 + 1e-20
    w1 = m1 / denom
    w2 = m2 / denom

    oh1 = jnp.where(lane == a1, 1.0, 0.0)
    oh2 = jnp.where(lane == a2, 1.0, 0.0)
    maskf = oh1 + oh2

    r_i = lax.broadcasted_iota(jnp.int32, (TB, TB), 0)
    c_i = lax.broadcasted_iota(jnp.int32, (TB, TB), 1)
    tril = jnp.where(c_i < r_i, 1.0, 0.0)
    pos_parts = []
    carry = jnp.zeros((1, E), jnp.float32)
    for bidx in range(NB):
        mb = maskf[bidx * TB:(bidx + 1) * TB]
        pb = lax.dot_general(tril, mb, (((1,), (0,)), ((), ())),
                             precision=lax.Precision.HIGHEST,
                             preferred_element_type=jnp.float32)
        pos_parts.append(pb + carry)
        carry = carry + jnp.sum(mb, axis=0, keepdims=True)
    pos = jnp.concatenate(pos_parts, axis=0)
    counts = carry
    ue_r = lax.broadcasted_iota(jnp.int32, (E, E), 0)
    ue_c = lax.broadcasted_iota(jnp.int32, (E, E), 1)
    triu = jnp.where(ue_r < ue_c, 1.0, 0.0)
    off = lax.dot_general(counts, triu, (((1,), (0,)), ((), ())),
                          precision=lax.Precision.HIGHEST,
                          preferred_element_type=jnp.float32)

    s0_ref[...] = jnp.sum((off + pos) * oh1, axis=1).astype(jnp.int32)
    s1_ref[...] = jnp.sum((off + pos) * oh2, axis=1).astype(jnp.int32)
    w0_ref[...] = jnp.broadcast_to(w1, (T, 16))
    w1_ref[...] = jnp.broadcast_to(w2, (T, 16))

    eye_r = jnp.where(ue_r == ue_c, 1.0, 0.0)
    counts_col = lax.dot_general(eye_r, counts, (((1,), (1,)), ((), ())),
                                 precision=lax.Precision.HIGHEST,
                                 preferred_element_type=jnp.float32)
    counts_ref[...] = counts_col


def _router_call(x, gate_weight):
    return pl.pallas_call(
        _router_body,
        grid=(1,),
        in_specs=[
            pl.BlockSpec((T, D), lambda i: (0, 0)),
            pl.BlockSpec((E, D), lambda i: (0, 0)),
        ],
        out_specs=[
            pl.BlockSpec((T,), lambda i: (0,)),
            pl.BlockSpec((T,), lambda i: (0,)),
            pl.BlockSpec((T, 16), lambda i: (0, 0)),
            pl.BlockSpec((T, 16), lambda i: (0, 0)),
            pl.BlockSpec((E, 1), lambda i: (0, 0)),
            pl.BlockSpec((T, DW), lambda i: (0, 0)),
        ],
        out_shape=[
            jax.ShapeDtypeStruct((T,), jnp.int32),
            jax.ShapeDtypeStruct((T,), jnp.int32),
            jax.ShapeDtypeStruct((T, 16), jnp.float32),
            jax.ShapeDtypeStruct((T, 16), jnp.float32),
            jax.ShapeDtypeStruct((E, 1), jnp.float32),
            jax.ShapeDtypeStruct((T, DW), jnp.int32),
        ],
    )(x, gate_weight)


def _dispatch_body(xp_hbm, s0_hbm, s1_hbm, xs_hbm,
                   idx0_v, idx1_v, rows_v, sem0, sem1):
    wid = lax.axis_index("s") * NC + lax.axis_index("c")
    base = wid * TPT
    l0 = pltpu.make_async_copy(s0_hbm.at[pl.ds(base, TPT)], idx0_v, sem0)
    l1 = pltpu.make_async_copy(s1_hbm.at[pl.ds(base, TPT)], idx1_v, sem1)
    l0.start(); l1.start()
    pltpu.sync_copy(xp_hbm.at[pl.ds(base, TPT)], rows_v)
    l0.wait(); l1.wait()
    c0 = pltpu.make_async_copy(rows_v, xs_hbm.at[idx0_v], sem0)
    c1 = pltpu.make_async_copy(rows_v, xs_hbm.at[idx1_v], sem1)
    c0.start(); c1.start(); c0.wait(); c1.wait()


def _dispatch_call(xp, slot0, slot1):
    f = functools.partial(
        pl.kernel,
        mesh=plsc.VectorSubcoreMesh(core_axis_name="c", subcore_axis_name="s"),
        out_type=jax.ShapeDtypeStruct((N, DW), jnp.int32),
        scratch_types=[
            pltpu.VMEM((TPT,), jnp.int32),
            pltpu.VMEM((TPT,), jnp.int32),
            pltpu.VMEM((TPT, DW), jnp.int32),
            pltpu.SemaphoreType.DMA,
            pltpu.SemaphoreType.DMA,
        ],
    )(_dispatch_body)
    return f(xp, slot0, slot1)


def _gmm_body(t_of, e_of, lo_a, hi_a, xs_ref, wg_ref, wu_ref, wd_ref,
              ys_ref):
    g = pl.program_id(0)
    lo = lo_a[g]
    hi = hi_a[g]
    w = xs_ref[...]
    xlo = lax.bitcast_convert_type(
        lax.shift_left(w, 16), jnp.float32).astype(jnp.bfloat16)
    xhi = lax.bitcast_convert_type(
        w & jnp.int32(-65536), jnp.float32).astype(jnp.bfloat16)
    wgb = wg_ref[0].astype(jnp.bfloat16)
    wub = wu_ref[0].astype(jnp.bfloat16)
    gg = (lax.dot_general(xlo, wgb[:, :DW], (((1,), (1,)), ((), ())),
                          preferred_element_type=jnp.float32)
          + lax.dot_general(xhi, wgb[:, DW:], (((1,), (1,)), ((), ())),
                            preferred_element_type=jnp.float32))
    uu = (lax.dot_general(xlo, wub[:, :DW], (((1,), (1,)), ((), ())),
                          preferred_element_type=jnp.float32)
          + lax.dot_general(xhi, wub[:, DW:], (((1,), (1,)), ((), ())),
                            preferred_element_type=jnp.float32))
    y = (gg / (1.0 + jnp.exp(-gg))) * uu
    dd = lax.dot_general(y.astype(jnp.bfloat16),
                         wd_ref[0].astype(jnp.bfloat16),
                         (((1,), (1,)), ((), ())),
                         preferred_element_type=jnp.float32)
    ddp = _pack_pairs(dd[:, :DW], dd[:, DW:])
    rows = lax.broadcasted_iota(jnp.int32, (M_TILE, DW), 0)
    mask = (rows >= lo) & (rows < hi)
    ys_ref[...] = jnp.where(mask, ddp, ys_ref[...])


def _gmm_call(t_of, e_of, lo, hi, xs, W_gate, W_up, W_down):
    grid_spec = pltpu.PrefetchScalarGridSpec(
        num_scalar_prefetch=4,
        grid=(G,),
        in_specs=[
            pl.BlockSpec((M_TILE, DW), lambda g, t, e, lo, hi: (t[g], 0)),
            pl.BlockSpec((1, FF, D), lambda g, t, e, lo, hi: (e[g], 0, 0)),
            pl.BlockSpec((1, FF, D), lambda g, t, e, lo, hi: (e[g], 0, 0)),
            pl.BlockSpec((1, D, FF), lambda g, t, e, lo, hi: (e[g], 0, 0)),
        ],
        out_specs=pl.BlockSpec((M_TILE, DW), lambda g, t, e, lo, hi: (t[g], 0)),
    )
    return pl.pallas_call(
        _gmm_body,
        grid_spec=grid_spec,
        out_shape=jax.ShapeDtypeStruct((N, DW), jnp.int32),
    )(t_of, e_of, lo, hi, xs, W_gate, W_up, W_down)


_CH = 32
_NW16 = DW // 16


def _combine_body(ys_hbm, s0_hbm, s1_hbm, w0_hbm, w1_hbm, out_hbm,
                  idx0_v, idx1_v, w0_v, w1_v,
                  r0a_v, r1a_v, r0b_v, r1b_v, o_v, sem0, sem1, semo):
    wid = lax.axis_index("s") * NC + lax.axis_index("c")
    tbase = wid * TPT
    lw0 = pltpu.make_async_copy(w0_hbm.at[pl.ds(tbase, TPT), :], w0_v, sem0)
    lw1 = pltpu.make_async_copy(w1_hbm.at[pl.ds(tbase, TPT), :], w1_v, sem1)
    li0 = pltpu.make_async_copy(s0_hbm.at[pl.ds(tbase, TPT)], idx0_v, sem0)
    li1 = pltpu.make_async_copy(s1_hbm.at[pl.ds(tbase, TPT)], idx1_v, sem1)
    lw0.start(); lw1.start(); li0.start(); li1.start()
    lw0.wait(); lw1.wait(); li0.wait(); li1.wait()
    bufs = ((r0a_v, r1a_v), (r0b_v, r1b_v))
    copies = []
    for ch in range(2):
        r0_v, r1_v = bufs[ch]
        g0 = pltpu.make_async_copy(
            ys_hbm.at[idx0_v.at[pl.ds(ch * _CH, _CH)]], r0_v, sem0)
        g1 = pltpu.make_async_copy(
            ys_hbm.at[idx1_v.at[pl.ds(ch * _CH, _CH)]], r1_v, sem1)
        g0.start(); g1.start()
        copies.append((g0, g1))
    for ch in range(2):
        r0_v, r1_v = bufs[ch]
        g0, g1 = copies[ch]
        g0.wait(); g1.wait()

        @plsc.parallel_loop(0, _CH, unroll=2)
        def _row(i):
            tok = ch * _CH + i
            wa = w0_v[tok, pl.ds(0, 16)]
            wb = w1_v[tok, pl.ds(0, 16)]
            for j in range(_NW16):
                sl = pl.ds(j * 16, 16)
                a = r0_v[i, sl]
                b = r1_v[i, sl]
                alo = lax.bitcast_convert_type(lax.shift_left(a, 16),
                                               jnp.float32)
                blo = lax.bitcast_convert_type(lax.shift_left(b, 16),
                                               jnp.float32)
                ahi = lax.bitcast_convert_type(a & jnp.int32(-65536),
                                               jnp.float32)
                bhi = lax.bitcast_convert_type(b & jnp.int32(-65536),
                                               jnp.float32)
                o_v[i, sl] = wa * alo + wb * blo
                o_v[i, pl.ds(DW + j * 16, 16)] = wa * ahi + wb * bhi

        pltpu.sync_copy(o_v, out_hbm.at[pl.ds(tbase + ch * _CH, _CH)])


def _combine_call(ys, slot0, slot1, w0, w1):
    f = functools.partial(
        pl.kernel,
        mesh=plsc.VectorSubcoreMesh(core_axis_name="c", subcore_axis_name="s"),
        out_type=jax.ShapeDtypeStruct((T, D), jnp.float32),
        scratch_types=[
            pltpu.VMEM((TPT,), jnp.int32),
            pltpu.VMEM((TPT,), jnp.int32),
            pltpu.VMEM((TPT, 16), jnp.float32),
            pltpu.VMEM((TPT, 16), jnp.float32),
            pltpu.VMEM((_CH, DW), jnp.int32),
            pltpu.VMEM((_CH, DW), jnp.int32),
            pltpu.VMEM((_CH, DW), jnp.int32),
            pltpu.VMEM((_CH, DW), jnp.int32),
            pltpu.VMEM((_CH, D), jnp.float32),
            pltpu.SemaphoreType.DMA,
            pltpu.SemaphoreType.DMA,
            pltpu.SemaphoreType.DMA,
        ],
    )(_combine_body)
    return f(ys, slot0, slot1, w0, w1)


def _schedule(counts_col):
    sizes = counts_col[:, 0].astype(jnp.int32)
    off = jnp.concatenate([jnp.zeros((1,), jnp.int32),
                           jnp.cumsum(sizes)[:-1].astype(jnp.int32)])
    ft = off // M_TILE
    lt = (off + sizes - 1) // M_TILE
    cnt = jnp.where(sizes > 0, lt - ft + 1, 1).astype(jnp.int32)
    start = jnp.concatenate([jnp.zeros((1,), jnp.int32),
                             jnp.cumsum(cnt)[:-1].astype(jnp.int32)])
    total = start[-1] + cnt[-1]
    g_ar = jnp.arange(G, dtype=jnp.int32)
    e_of = (jnp.sum(start[None, :] <= g_ar[:, None], axis=1) - 1
            ).astype(jnp.int32)
    t_raw = ft[e_of] + (g_ar - start[e_of])
    valid = g_ar < total
    t_of = jnp.clip(t_raw, 0, NT - 1).astype(jnp.int32)
    glo = jnp.maximum(off[e_of], t_of * M_TILE)
    ghi = jnp.minimum(off[e_of] + sizes[e_of], (t_of + 1) * M_TILE)
    lo = jnp.where(valid, glo - t_of * M_TILE, 0).astype(jnp.int32)
    hi = jnp.where(valid, ghi - t_of * M_TILE, 0).astype(jnp.int32)
    lo = jnp.clip(lo, 0, M_TILE)
    hi = jnp.clip(hi, 0, M_TILE)
    return t_of, e_of, lo, hi


def kernel(hidden_states, gate_weight, W_gate, W_up, W_down):
    b, s, d = hidden_states.shape
    x = hidden_states.reshape(-1, d)

    slot0, slot1, w0, w1, counts_col, xp = _router_call(x, gate_weight)
    t_of, e_of, lo, hi = _schedule(counts_col)

    xs = _dispatch_call(xp, slot0, slot1)
    ys = _gmm_call(t_of, e_of, lo, hi, xs, W_gate, W_up, W_down)
    out = _combine_call(ys, slot0, slot1, w0, w1)
    return out.reshape(b, s, d)


# speedup vs baseline: 1.0009x; 1.0009x over previous
import functools

import jax
import jax.numpy as jnp
from jax import lax
from jax.experimental import pallas as pl
from jax.experimental.pallas import tpu as pltpu
from jax.experimental.pallas import tpu_sc as plsc

E = 8
K = 2
D = 1024
FF = 512
T = 2048
N = T * K
M_TILE = 1024
NT = N // M_TILE
G = NT + E - 1
TB = 256
NB = T // TB

NC = 2
NS = 16
NW = NC * NS
TPT = T // NW
DW = D // 2


def _pack_pairs(lo_f32, hi_f32):
    lob = lax.bitcast_convert_type(lo_f32, jnp.int32)
    hib = lax.bitcast_convert_type(hi_f32, jnp.int32)
    lo16 = lax.shift_right_logical(lob + 0x8000, 16)
    hi16 = (hib + 0x8000) & jnp.int32(-65536)
    return hi16 | lo16


def _router_body(x_ref, gw_ref, s0_ref, s1_ref, w0_ref, w1_ref, counts_ref,
                 xp_ref):
    x = x_ref[...]
    xp_ref[...] = _pack_pairs(x[:, :DW], x[:, DW:])
    gw = gw_ref[...]
    logits = lax.dot_general(x, gw, (((1,), (1,)), ((), ())),
                             preferred_element_type=jnp.float32)
    m = jnp.max(logits, axis=1, keepdims=True)
    ex = jnp.exp(logits - m)
    p = ex / jnp.sum(ex, axis=1, keepdims=True)
    lane = lax.broadcasted_iota(jnp.int32, p.shape, 1)
    m1 = jnp.max(p, axis=1, keepdims=True)
    a1 = jnp.min(jnp.where(p == m1, lane, E), axis=1, keepdims=True)
    p2 = jnp.where(lane == a1, -jnp.inf, p)
    m2 = jnp.max(p2, axis=1, keepdims=True)
    a2 = jnp.min(jnp.where(p2 == m2, lane, E), axis=1, keepdims=True)
    denom = m1 + m2 + 1e-20
    w1 = m1 / denom
    w2 = m2 / denom

    oh1 = jnp.where(lane == a1, 1.0, 0.0)
    oh2 = jnp.where(lane == a2, 1.0, 0.0)
    maskf = oh1 + oh2

    r_i = lax.broadcasted_iota(jnp.int32, (TB, TB), 0)
    c_i = lax.broadcasted_iota(jnp.int32, (TB, TB), 1)
    tril = jnp.where(c_i < r_i, 1.0, 0.0)
    pos_parts = []
    carry = jnp.zeros((1, E), jnp.float32)
    for bidx in range(NB):
        mb = maskf[bidx * TB:(bidx + 1) * TB]
        pb = lax.dot_general(tril, mb, (((1,), (0,)), ((), ())),
                             precision=lax.Precision.HIGHEST,
                             preferred_element_type=jnp.float32)
        pos_parts.append(pb + carry)
        carry = carry + jnp.sum(mb, axis=0, keepdims=True)
    pos = jnp.concatenate(pos_parts, axis=0)
    counts = carry
    ue_r = lax.broadcasted_iota(jnp.int32, (E, E), 0)
    ue_c = lax.broadcasted_iota(jnp.int32, (E, E), 1)
    triu = jnp.where(ue_r < ue_c, 1.0, 0.0)
    off = lax.dot_general(counts, triu, (((1,), (0,)), ((), ())),
                          precision=lax.Precision.HIGHEST,
                          preferred_element_type=jnp.float32)

    s0_ref[...] = jnp.sum((off + pos) * oh1, axis=1).astype(jnp.int32)
    s1_ref[...] = jnp.sum((off + pos) * oh2, axis=1).astype(jnp.int32)
    w0_ref[...] = jnp.broadcast_to(w1, (T, 16))
    w1_ref[...] = jnp.broadcast_to(w2, (T, 16))

    eye_r = jnp.where(ue_r == ue_c, 1.0, 0.0)
    counts_col = lax.dot_general(eye_r, counts, (((1,), (1,)), ((), ())),
                                 precision=lax.Precision.HIGHEST,
                                 preferred_element_type=jnp.float32)
    counts_ref[...] = counts_col


def _router_call(x, gate_weight):
    return pl.pallas_call(
        _router_body,
        grid=(1,),
        in_specs=[
            pl.BlockSpec((T, D), lambda i: (0, 0)),
            pl.BlockSpec((E, D), lambda i: (0, 0)),
        ],
        out_specs=[
            pl.BlockSpec((T,), lambda i: (0,)),
            pl.BlockSpec((T,), lambda i: (0,)),
            pl.BlockSpec((T, 16), lambda i: (0, 0)),
            pl.BlockSpec((T, 16), lambda i: (0, 0)),
            pl.BlockSpec((E, 1), lambda i: (0, 0)),
            pl.BlockSpec((T, DW), lambda i: (0, 0)),
        ],
        out_shape=[
            jax.ShapeDtypeStruct((T,), jnp.int32),
            jax.ShapeDtypeStruct((T,), jnp.int32),
            jax.ShapeDtypeStruct((T, 16), jnp.float32),
            jax.ShapeDtypeStruct((T, 16), jnp.float32),
            jax.ShapeDtypeStruct((E, 1), jnp.float32),
            jax.ShapeDtypeStruct((T, DW), jnp.int32),
        ],
    )(x, gate_weight)


def _dispatch_body(xp_hbm, s0_hbm, s1_hbm, xs_hbm,
                   idx0_v, idx1_v, rows_v, sem0, sem1):
    wid = lax.axis_index("s") * NC + lax.axis_index("c")
    base = wid * TPT
    l0 = pltpu.make_async_copy(s0_hbm.at[pl.ds(base, TPT)], idx0_v, sem0)
    l1 = pltpu.make_async_copy(s1_hbm.at[pl.ds(base, TPT)], idx1_v, sem1)
    l0.start(); l1.start()
    pltpu.sync_copy(xp_hbm.at[pl.ds(base, TPT)], rows_v)
    l0.wait(); l1.wait()
    c0 = pltpu.make_async_copy(rows_v, xs_hbm.at[idx0_v], sem0)
    c1 = pltpu.make_async_copy(rows_v, xs_hbm.at[idx1_v], sem1)
    c0.start(); c1.start(); c0.wait(); c1.wait()


def _dispatch_call(xp, slot0, slot1):
    f = functools.partial(
        pl.kernel,
        mesh=plsc.VectorSubcoreMesh(core_axis_name="c", subcore_axis_name="s"),
        out_type=jax.ShapeDtypeStruct((N, DW), jnp.int32),
        scratch_types=[
            pltpu.VMEM((TPT,), jnp.int32),
            pltpu.VMEM((TPT,), jnp.int32),
            pltpu.VMEM((TPT, DW), jnp.int32),
            pltpu.SemaphoreType.DMA,
            pltpu.SemaphoreType.DMA,
        ],
    )(_dispatch_body)
    return f(xp, slot0, slot1)


def _gmm_body(t_of, e_of, lo_a, hi_a, xs_ref, wg_ref, wu_ref, wd_ref,
              ys_ref):
    g = pl.program_id(0)
    lo = lo_a[g]
    hi = hi_a[g]
    w = xs_ref[...]
    xlo = lax.bitcast_convert_type(
        lax.shift_left(w, 16), jnp.float32).astype(jnp.bfloat16)
    xhi = lax.bitcast_convert_type(
        w & jnp.int32(-65536), jnp.float32).astype(jnp.bfloat16)
    wgb = wg_ref[0].astype(jnp.bfloat16)
    wub = wu_ref[0].astype(jnp.bfloat16)
    gg = (lax.dot_general(xlo, wgb[:, :DW], (((1,), (1,)), ((), ())),
                          preferred_element_type=jnp.float32)
          + lax.dot_general(xhi, wgb[:, DW:], (((1,), (1,)), ((), ())),
                            preferred_element_type=jnp.float32))
    uu = (lax.dot_general(xlo, wub[:, :DW], (((1,), (1,)), ((), ())),
                          preferred_element_type=jnp.float32)
          + lax.dot_general(xhi, wub[:, DW:], (((1,), (1,)), ((), ())),
                            preferred_element_type=jnp.float32))
    y = (gg / (1.0 + jnp.exp(-gg))) * uu
    dd = lax.dot_general(y.astype(jnp.bfloat16),
                         wd_ref[0].astype(jnp.bfloat16),
                         (((1,), (1,)), ((), ())),
                         preferred_element_type=jnp.float32)
    ddp = _pack_pairs(dd[:, :DW], dd[:, DW:])
    rows = lax.broadcasted_iota(jnp.int32, (M_TILE, DW), 0)
    mask = (rows >= lo) & (rows < hi)
    ys_ref[...] = jnp.where(mask, ddp, ys_ref[...])


def _gmm_call(t_of, e_of, lo, hi, xs, W_gate, W_up, W_down):
    grid_spec = pltpu.PrefetchScalarGridSpec(
        num_scalar_prefetch=4,
        grid=(G,),
        in_specs=[
            pl.BlockSpec((M_TILE, DW), lambda g, t, e, lo, hi: (t[g], 0)),
            pl.BlockSpec((1, FF, D), lambda g, t, e, lo, hi: (e[g], 0, 0)),
            pl.BlockSpec((1, FF, D), lambda g, t, e, lo, hi: (e[g], 0, 0)),
            pl.BlockSpec((1, D, FF), lambda g, t, e, lo, hi: (e[g], 0, 0)),
        ],
        out_specs=pl.BlockSpec((M_TILE, DW), lambda g, t, e, lo, hi: (t[g], 0)),
    )
    return pl.pallas_call(
        _gmm_body,
        grid_spec=grid_spec,
        out_shape=jax.ShapeDtypeStruct((N, DW), jnp.int32),
    )(t_of, e_of, lo, hi, xs, W_gate, W_up, W_down)


_CH = 32
_NW16 = DW // 16


def _combine_body(ys_hbm, s0_hbm, s1_hbm, w0_hbm, w1_hbm, out_hbm,
                  idx0_v, idx1_v, w0_v, w1_v,
                  r0a_v, r1a_v, r0b_v, r1b_v, o_v, sem0, sem1, semo):
    wid = lax.axis_index("s") * NC + lax.axis_index("c")
    tbase = wid * TPT
    lw0 = pltpu.make_async_copy(w0_hbm.at[pl.ds(tbase, TPT), :], w0_v, sem0)
    lw1 = pltpu.make_async_copy(w1_hbm.at[pl.ds(tbase, TPT), :], w1_v, sem1)
    li0 = pltpu.make_async_copy(s0_hbm.at[pl.ds(tbase, TPT)], idx0_v, sem0)
    li1 = pltpu.make_async_copy(s1_hbm.at[pl.ds(tbase, TPT)], idx1_v, sem1)
    lw0.start(); lw1.start(); li0.start(); li1.start()
    lw0.wait(); lw1.wait(); li0.wait(); li1.wait()
    bufs = ((r0a_v, r1a_v), (r0b_v, r1b_v))
    copies = []
    for ch in range(2):
        r0_v, r1_v = bufs[ch]
        g0 = pltpu.make_async_copy(
            ys_hbm.at[idx0_v.at[pl.ds(ch * _CH, _CH)]], r0_v, sem0)
        g1 = pltpu.make_async_copy(
            ys_hbm.at[idx1_v.at[pl.ds(ch * _CH, _CH)]], r1_v, sem1)
        g0.start(); g1.start()
        copies.append((g0, g1))
    for ch in range(2):
        r0_v, r1_v = bufs[ch]
        g0, g1 = copies[ch]
        g0.wait(); g1.wait()

        @plsc.parallel_loop(0, _CH, unroll=2)
        def _row(i):
            tok = ch * _CH + i
            wa = w0_v[tok, pl.ds(0, 16)]
            wb = w1_v[tok, pl.ds(0, 16)]
            for j in range(_NW16):
                sl = pl.ds(j * 16, 16)
                a = r0_v[i, sl]
                b = r1_v[i, sl]
                alo = lax.bitcast_convert_type(lax.shift_left(a, 16),
                                               jnp.float32)
                blo = lax.bitcast_convert_type(lax.shift_left(b, 16),
                                               jnp.float32)
                ahi = lax.bitcast_convert_type(a & jnp.int32(-65536),
                                               jnp.float32)
                bhi = lax.bitcast_convert_type(b & jnp.int32(-65536),
                                               jnp.float32)
                o_v[i, sl] = wa * alo + wb * blo
                o_v[i, pl.ds(DW + j * 16, 16)] = wa * ahi + wb * bhi

        pltpu.sync_copy(o_v, out_hbm.at[pl.ds(tbase + ch * _CH, _CH)])


def _combine_call(ys, slot0, slot1, w0, w1):
    f = functools.partial(
        pl.kernel,
        mesh=plsc.VectorSubcoreMesh(core_axis_name="c", subcore_axis_name="s"),
        out_type=jax.ShapeDtypeStruct((T, D), jnp.float32),
        scratch_types=[
            pltpu.VMEM((TPT,), jnp.int32),
            pltpu.VMEM((TPT,), jnp.int32),
            pltpu.VMEM((TPT, 16), jnp.float32),
            pltpu.VMEM((TPT, 16), jnp.float32),
            pltpu.VMEM((_CH, DW), jnp.int32),
            pltpu.VMEM((_CH, DW), jnp.int32),
            pltpu.VMEM((_CH, DW), jnp.int32),
            pltpu.VMEM((_CH, DW), jnp.int32),
            pltpu.VMEM((_CH, D), jnp.float32),
            pltpu.SemaphoreType.DMA,
            pltpu.SemaphoreType.DMA,
            pltpu.SemaphoreType.DMA,
        ],
    )(_combine_body)
    return f(ys, slot0, slot1, w0, w1)


def _schedule(counts_col):
    sizes = counts_col[:, 0].astype(jnp.int32)
    off = jnp.concatenate([jnp.zeros((1,), jnp.int32),
                           jnp.cumsum(sizes)[:-1].astype(jnp.int32)])
    ft = off // M_TILE
    lt = (off + sizes - 1) // M_TILE
    cnt = jnp.where(sizes > 0, lt - ft + 1, 1).astype(jnp.int32)
    start = jnp.concatenate([jnp.zeros((1,), jnp.int32),
                             jnp.cumsum(cnt)[:-1].astype(jnp.int32)])
    total = start[-1] + cnt[-1]
    g_ar = jnp.arange(G, dtype=jnp.int32)
    e_of = (jnp.sum(start[None, :] <= g_ar[:, None], axis=1) - 1
            ).astype(jnp.int32)
    t_raw = ft[e_of] + (g_ar - start[e_of])
    valid = g_ar < total
    t_of = jnp.clip(t_raw, 0, NT - 1).astype(jnp.int32)
    glo = jnp.maximum(off[e_of], t_of * M_TILE)
    ghi = jnp.minimum(off[e_of] + sizes[e_of], (t_of + 1) * M_TILE)
    lo = jnp.where(valid, glo - t_of * M_TILE, 0).astype(jnp.int32)
    hi = jnp.where(valid, ghi - t_of * M_TILE, 0).astype(jnp.int32)
    lo = jnp.clip(lo, 0, M_TILE)
    hi = jnp.clip(hi, 0, M_TILE)
    return t_of, e_of, lo, hi


def kernel(hidden_states, gate_weight, W_gate, W_up, W_down):
    b, s, d = hidden_states.shape
    x = hidden_states.reshape(-1, d)

    slot0, slot1, w0, w1, counts_col, xp = _router_call(x, gate_weight)
    t_of, e_of, lo, hi = _schedule(counts_col)

    xs = _dispatch_call(xp, slot0, slot1)
    ys = _gmm_call(t_of, e_of, lo, hi, xs, W_gate, W_up, W_down)
    out = _combine_call(ys, slot0, slot1, w0, w1)
    return out.reshape(b, s, d)


# speedup vs baseline: 1.0511x; 1.0502x over previous
import functools

import jax
import jax.numpy as jnp
from jax import lax
from jax.experimental import pallas as pl
from jax.experimental.pallas import tpu as pltpu
from jax.experimental.pallas import tpu_sc as plsc

E = 8
K = 2
D = 1024
FF = 512
T = 2048
N = T * K
M_TILE = 512
NT = N // M_TILE
G = NT + E - 1
TB = 256
NB = T // TB

NC = 2
NS = 16
NW = NC * NS
TPT = T // NW
DW = D // 2


def _pack_pairs(lo_f32, hi_f32):
    lob = lax.bitcast_convert_type(lo_f32, jnp.int32)
    hib = lax.bitcast_convert_type(hi_f32, jnp.int32)
    lo16 = lax.shift_right_logical(lob + 0x8000, 16)
    hi16 = (hib + 0x8000) & jnp.int32(-65536)
    return hi16 | lo16


def _router_body(x_ref, gw_ref, s0_ref, s1_ref, w0_ref, w1_ref, counts_ref,
                 xp_ref):
    x = x_ref[...]
    xp_ref[...] = _pack_pairs(x[:, :DW], x[:, DW:])
    gw = gw_ref[...]
    logits = lax.dot_general(x, gw, (((1,), (1,)), ((), ())),
                             preferred_element_type=jnp.float32)
    m = jnp.max(logits, axis=1, keepdims=True)
    ex = jnp.exp(logits - m)
    p = ex / jnp.sum(ex, axis=1, keepdims=True)
    lane = lax.broadcasted_iota(jnp.int32, p.shape, 1)
    m1 = jnp.max(p, axis=1, keepdims=True)
    a1 = jnp.min(jnp.where(p == m1, lane, E), axis=1, keepdims=True)
    p2 = jnp.where(lane == a1, -jnp.inf, p)
    m2 = jnp.max(p2, axis=1, keepdims=True)
    a2 = jnp.min(jnp.where(p2 == m2, lane, E), axis=1, keepdims=True)
    denom = m1 + m2 + 1e-20
    w1 = m1 / denom
    w2 = m2 / denom

    oh1 = jnp.where(lane == a1, 1.0, 0.0)
    oh2 = jnp.where(lane == a2, 1.0, 0.0)
    maskf = oh1 + oh2

    r_i = lax.broadcasted_iota(jnp.int32, (TB, TB), 0)
    c_i = lax.broadcasted_iota(jnp.int32, (TB, TB), 1)
    tril = jnp.where(c_i < r_i, 1.0, 0.0)
    pos_parts = []
    carry = jnp.zeros((1, E), jnp.float32)
    for bidx in range(NB):
        mb = maskf[bidx * TB:(bidx + 1) * TB]
        pb = lax.dot_general(tril, mb, (((1,), (0,)), ((), ())),
                             precision=lax.Precision.HIGHEST,
                             preferred_element_type=jnp.float32)
        pos_parts.append(pb + carry)
        carry = carry + jnp.sum(mb, axis=0, keepdims=True)
    pos = jnp.concatenate(pos_parts, axis=0)
    counts = carry
    ue_r = lax.broadcasted_iota(jnp.int32, (E, E), 0)
    ue_c = lax.broadcasted_iota(jnp.int32, (E, E), 1)
    triu = jnp.where(ue_r < ue_c, 1.0, 0.0)
    off = lax.dot_general(counts, triu, (((1,), (0,)), ((), ())),
                          precision=lax.Precision.HIGHEST,
                          preferred_element_type=jnp.float32)

    s0_ref[...] = jnp.sum((off + pos) * oh1, axis=1).astype(jnp.int32)
    s1_ref[...] = jnp.sum((off + pos) * oh2, axis=1).astype(jnp.int32)
    w0_ref[...] = jnp.broadcast_to(w1, (T, 16))
    w1_ref[...] = jnp.broadcast_to(w2, (T, 16))

    eye_r = jnp.where(ue_r == ue_c, 1.0, 0.0)
    counts_col = lax.dot_general(eye_r, counts, (((1,), (1,)), ((), ())),
                                 precision=lax.Precision.HIGHEST,
                                 preferred_element_type=jnp.float32)
    counts_ref[...] = counts_col


def _router_call(x, gate_weight):
    return pl.pallas_call(
        _router_body,
        grid=(1,),
        in_specs=[
            pl.BlockSpec((T, D), lambda i: (0, 0)),
            pl.BlockSpec((E, D), lambda i: (0, 0)),
        ],
        out_specs=[
            pl.BlockSpec((T,), lambda i: (0,)),
            pl.BlockSpec((T,), lambda i: (0,)),
            pl.BlockSpec((T, 16), lambda i: (0, 0)),
            pl.BlockSpec((T, 16), lambda i: (0, 0)),
            pl.BlockSpec((E, 1), lambda i: (0, 0)),
            pl.BlockSpec((T, DW), lambda i: (0, 0)),
        ],
        out_shape=[
            jax.ShapeDtypeStruct((T,), jnp.int32),
            jax.ShapeDtypeStruct((T,), jnp.int32),
            jax.ShapeDtypeStruct((T, 16), jnp.float32),
            jax.ShapeDtypeStruct((T, 16), jnp.float32),
            jax.ShapeDtypeStruct((E, 1), jnp.float32),
            jax.ShapeDtypeStruct((T, DW), jnp.int32),
        ],
    )(x, gate_weight)


def _dispatch_body(xp_hbm, s0_hbm, s1_hbm, xs_hbm,
                   idx0_v, idx1_v, rows_v, sem0, sem1):
    wid = lax.axis_index("s") * NC + lax.axis_index("c")
    base = wid * TPT
    l0 = pltpu.make_async_copy(s0_hbm.at[pl.ds(base, TPT)], idx0_v, sem0)
    l1 = pltpu.make_async_copy(s1_hbm.at[pl.ds(base, TPT)], idx1_v, sem1)
    l0.start(); l1.start()
    pltpu.sync_copy(xp_hbm.at[pl.ds(base, TPT)], rows_v)
    l0.wait(); l1.wait()
    c0 = pltpu.make_async_copy(rows_v, xs_hbm.at[idx0_v], sem0)
    c1 = pltpu.make_async_copy(rows_v, xs_hbm.at[idx1_v], sem1)
    c0.start(); c1.start(); c0.wait(); c1.wait()


def _dispatch_call(xp, slot0, slot1):
    f = functools.partial(
        pl.kernel,
        mesh=plsc.VectorSubcoreMesh(core_axis_name="c", subcore_axis_name="s"),
        out_type=jax.ShapeDtypeStruct((N, DW), jnp.int32),
        scratch_types=[
            pltpu.VMEM((TPT,), jnp.int32),
            pltpu.VMEM((TPT,), jnp.int32),
            pltpu.VMEM((TPT, DW), jnp.int32),
            pltpu.SemaphoreType.DMA,
            pltpu.SemaphoreType.DMA,
        ],
    )(_dispatch_body)
    return f(xp, slot0, slot1)


def _gmm_body(t_of, e_of, lo_a, hi_a, xs_ref, wg_ref, wu_ref, wd_ref,
              ys_ref):
    g = pl.program_id(0)
    lo = lo_a[g]
    hi = hi_a[g]
    w = xs_ref[...]
    xlo = lax.bitcast_convert_type(
        lax.shift_left(w, 16), jnp.float32).astype(jnp.bfloat16)
    xhi = lax.bitcast_convert_type(
        w & jnp.int32(-65536), jnp.float32).astype(jnp.bfloat16)
    wgb = wg_ref[0].astype(jnp.bfloat16)
    wub = wu_ref[0].astype(jnp.bfloat16)
    gg = (lax.dot_general(xlo, wgb[:, :DW], (((1,), (1,)), ((), ())),
                          preferred_element_type=jnp.float32)
          + lax.dot_general(xhi, wgb[:, DW:], (((1,), (1,)), ((), ())),
                            preferred_element_type=jnp.float32))
    uu = (lax.dot_general(xlo, wub[:, :DW], (((1,), (1,)), ((), ())),
                          preferred_element_type=jnp.float32)
          + lax.dot_general(xhi, wub[:, DW:], (((1,), (1,)), ((), ())),
                            preferred_element_type=jnp.float32))
    y = (gg / (1.0 + jnp.exp(-gg))) * uu
    dd = lax.dot_general(y.astype(jnp.bfloat16),
                         wd_ref[0].astype(jnp.bfloat16),
                         (((1,), (1,)), ((), ())),
                         preferred_element_type=jnp.float32)
    ddp = _pack_pairs(dd[:, :DW], dd[:, DW:])
    rows = lax.broadcasted_iota(jnp.int32, (M_TILE, DW), 0)
    mask = (rows >= lo) & (rows < hi)
    ys_ref[...] = jnp.where(mask, ddp, ys_ref[...])


def _gmm_call(t_of, e_of, lo, hi, xs, W_gate, W_up, W_down):
    grid_spec = pltpu.PrefetchScalarGridSpec(
        num_scalar_prefetch=4,
        grid=(G,),
        in_specs=[
            pl.BlockSpec((M_TILE, DW), lambda g, t, e, lo, hi: (t[g], 0)),
            pl.BlockSpec((1, FF, D), lambda g, t, e, lo, hi: (e[g], 0, 0)),
            pl.BlockSpec((1, FF, D), lambda g, t, e, lo, hi: (e[g], 0, 0)),
            pl.BlockSpec((1, D, FF), lambda g, t, e, lo, hi: (e[g], 0, 0)),
        ],
        out_specs=pl.BlockSpec((M_TILE, DW), lambda g, t, e, lo, hi: (t[g], 0)),
    )
    return pl.pallas_call(
        _gmm_body,
        grid_spec=grid_spec,
        out_shape=jax.ShapeDtypeStruct((N, DW), jnp.int32),
    )(t_of, e_of, lo, hi, xs, W_gate, W_up, W_down)


_CH = 32
_NW16 = DW // 16


def _combine_body(ys_hbm, s0_hbm, s1_hbm, w0_hbm, w1_hbm, out_hbm,
                  idx0_v, idx1_v, w0_v, w1_v,
                  r0a_v, r1a_v, r0b_v, r1b_v, o_v, sem0, sem1, semo):
    wid = lax.axis_index("s") * NC + lax.axis_index("c")
    tbase = wid * TPT
    lw0 = pltpu.make_async_copy(w0_hbm.at[pl.ds(tbase, TPT), :], w0_v, sem0)
    lw1 = pltpu.make_async_copy(w1_hbm.at[pl.ds(tbase, TPT), :], w1_v, sem1)
    li0 = pltpu.make_async_copy(s0_hbm.at[pl.ds(tbase, TPT)], idx0_v, sem0)
    li1 = pltpu.make_async_copy(s1_hbm.at[pl.ds(tbase, TPT)], idx1_v, sem1)
    lw0.start(); lw1.start(); li0.start(); li1.start()
    lw0.wait(); lw1.wait(); li0.wait(); li1.wait()
    bufs = ((r0a_v, r1a_v), (r0b_v, r1b_v))
    copies = []
    for ch in range(2):
        r0_v, r1_v = bufs[ch]
        g0 = pltpu.make_async_copy(
            ys_hbm.at[idx0_v.at[pl.ds(ch * _CH, _CH)]], r0_v, sem0)
        g1 = pltpu.make_async_copy(
            ys_hbm.at[idx1_v.at[pl.ds(ch * _CH, _CH)]], r1_v, sem1)
        g0.start(); g1.start()
        copies.append((g0, g1))
    for ch in range(2):
        r0_v, r1_v = bufs[ch]
        g0, g1 = copies[ch]
        g0.wait(); g1.wait()

        @plsc.parallel_loop(0, _CH, unroll=4)
        def _row(i):
            tok = ch * _CH + i
            wa = w0_v[tok, pl.ds(0, 16)]
            wb = w1_v[tok, pl.ds(0, 16)]
            for j in range(_NW16):
                sl = pl.ds(j * 16, 16)
                a = r0_v[i, sl]
                b = r1_v[i, sl]
                alo = lax.bitcast_convert_type(lax.shift_left(a, 16),
                                               jnp.float32)
                blo = lax.bitcast_convert_type(lax.shift_left(b, 16),
                                               jnp.float32)
                ahi = lax.bitcast_convert_type(a & jnp.int32(-65536),
                                               jnp.float32)
                bhi = lax.bitcast_convert_type(b & jnp.int32(-65536),
                                               jnp.float32)
                o_v[i, sl] = wa * alo + wb * blo
                o_v[i, pl.ds(DW + j * 16, 16)] = wa * ahi + wb * bhi

        pltpu.sync_copy(o_v, out_hbm.at[pl.ds(tbase + ch * _CH, _CH)])


def _combine_call(ys, slot0, slot1, w0, w1):
    f = functools.partial(
        pl.kernel,
        mesh=plsc.VectorSubcoreMesh(core_axis_name="c", subcore_axis_name="s"),
        out_type=jax.ShapeDtypeStruct((T, D), jnp.float32),
        scratch_types=[
            pltpu.VMEM((TPT,), jnp.int32),
            pltpu.VMEM((TPT,), jnp.int32),
            pltpu.VMEM((TPT, 16), jnp.float32),
            pltpu.VMEM((TPT, 16), jnp.float32),
            pltpu.VMEM((_CH, DW), jnp.int32),
            pltpu.VMEM((_CH, DW), jnp.int32),
            pltpu.VMEM((_CH, DW), jnp.int32),
            pltpu.VMEM((_CH, DW), jnp.int32),
            pltpu.VMEM((_CH, D), jnp.float32),
            pltpu.SemaphoreType.DMA,
            pltpu.SemaphoreType.DMA,
            pltpu.SemaphoreType.DMA,
        ],
    )(_combine_body)
    return f(ys, slot0, slot1, w0, w1)


def _schedule(counts_col):
    sizes = counts_col[:, 0].astype(jnp.int32)
    off = jnp.concatenate([jnp.zeros((1,), jnp.int32),
                           jnp.cumsum(sizes)[:-1].astype(jnp.int32)])
    ft = off // M_TILE
    lt = (off + sizes - 1) // M_TILE
    cnt = jnp.where(sizes > 0, lt - ft + 1, 1).astype(jnp.int32)
    start = jnp.concatenate([jnp.zeros((1,), jnp.int32),
                             jnp.cumsum(cnt)[:-1].astype(jnp.int32)])
    total = start[-1] + cnt[-1]
    g_ar = jnp.arange(G, dtype=jnp.int32)
    e_of = (jnp.sum(start[None, :] <= g_ar[:, None], axis=1) - 1
            ).astype(jnp.int32)
    t_raw = ft[e_of] + (g_ar - start[e_of])
    valid = g_ar < total
    t_of = jnp.clip(t_raw, 0, NT - 1).astype(jnp.int32)
    glo = jnp.maximum(off[e_of], t_of * M_TILE)
    ghi = jnp.minimum(off[e_of] + sizes[e_of], (t_of + 1) * M_TILE)
    lo = jnp.where(valid, glo - t_of * M_TILE, 0).astype(jnp.int32)
    hi = jnp.where(valid, ghi - t_of * M_TILE, 0).astype(jnp.int32)
    lo = jnp.clip(lo, 0, M_TILE)
    hi = jnp.clip(hi, 0, M_TILE)
    return t_of, e_of, lo, hi


def kernel(hidden_states, gate_weight, W_gate, W_up, W_down):
    b, s, d = hidden_states.shape
    x = hidden_states.reshape(-1, d)

    slot0, slot1, w0, w1, counts_col, xp = _router_call(x, gate_weight)
    t_of, e_of, lo, hi = _schedule(counts_col)

    xs = _dispatch_call(xp, slot0, slot1)
    ys = _gmm_call(t_of, e_of, lo, hi, xs, W_gate, W_up, W_down)
    out = _combine_call(ys, slot0, slot1, w0, w1)
    return out.reshape(b, s, d)
